# baseline (speedup 1.0000x reference)
import functools

import jax
import jax.numpy as jnp
from jax import lax
from jax.experimental import pallas as pl
from jax.experimental.pallas import tpu as pltpu
from jax.experimental.pallas import tpu_sc as plsc

N = 10000
E = 160000
C = 128

NSUB = 16
NCORE = 2
NPAD = 10240
RPT = NPAD // NSUB
HC = C // NCORE
EPT = E // NSUB
BATCH = 128
NB = (EPT + BATCH - 1) // BATCH
EPAD = 10240
NB = EPAD // BATCH
PAD_NODE = NPAD - 1
NSLOT = 11
CH = 128
NBUF = 4
FB = 4 * BATCH
NFB = NB // 4

STAGE1_STORES = {1: 1, 2: 2, 4: 3, 8: 4}
STAGE1_EMITS = {2: 1, 4: 2, 8: 3, 16: 4}
STAGE2_STORES = ({2: 5, 4: 6, 8: 8}, {4: 7, 8: 9}, {8: 10})
STAGE2_EMITS = ({4: 5, 8: 6, 16: 8}, {8: 7, 16: 9}, {16: 10})


def _sc_scatter(x_pad, epad):
    mesh = plsc.VectorSubcoreMesh(core_axis_name="c", subcore_axis_name="s")

    @functools.partial(
        pl.kernel,
        out_type=[
            jax.ShapeDtypeStruct((NSLOT, NCORE, NPAD, HC), jnp.float32),
            jax.ShapeDtypeStruct((NCORE * NPAD, HC), jnp.float32),
        ],
        mesh=mesh,
        compiler_params=pltpu.CompilerParams(use_tc_tiling_on_sc=False),
        scratch_types=[
            pltpu.VMEM_SHARED((NPAD, HC), jnp.float32),
            pltpu.VMEM((NFB, FB), jnp.int32),
            pltpu.VMEM((NFB, FB), jnp.int32),
            pltpu.VMEM((FB, HC), jnp.float32),
            pltpu.VMEM((CH, HC), jnp.float32),
            pltpu.VMEM((CH, HC), jnp.float32),
            pltpu.VMEM((CH, HC), jnp.float32),
            pltpu.VMEM((RPT, 16), jnp.float32),
            pltpu.SemaphoreType.DMA((NBUF,)),
            pltpu.SemaphoreType.DMA((NBUF,)),
        ],
    )
    def k(x_hbm, e_hbm, feat_hbm, xst_hbm, acc_s,
          row_v, col_v, gfat, buf, buf2, ebuf, dinvb, gsem, ssem):
        c = lax.axis_index("c")
        s = lax.axis_index("s")
        base = s * RPT
        NG = HC // 16
        zero16 = jnp.zeros((16,), jnp.float32)
        one16 = jnp.ones((16,), jnp.float32)

        pltpu.sync_copy(e_hbm.at[0, s], row_v)
        pltpu.sync_copy(e_hbm.at[1, s], col_v)

        coff = c * NPAD

        def off_body(i, _):
            r = i // (FB // 16)
            g = i % (FB // 16)
            row_v[r, pl.ds(g * 16, 16)] = row_v[r, pl.ds(g * 16, 16)] + coff
            return 0
        lax.fori_loop(0, NFB * (FB // 16), off_body, 0)

        def fill_ones(i, _):
            gfat[i // NG, pl.ds((i % NG) * 16, 16)] = one16
            return 0
        lax.fori_loop(0, FB * NG, fill_ones, 0)

        def fill_zero(i, _):
            buf[i // NG, pl.ds((i % NG) * 16, 16)] = zero16
            return 0
        lax.fori_loop(0, CH * NG, fill_zero, 0)
        for h in range(RPT // CH):
            pltpu.sync_copy(buf, acc_s.at[pl.ds(base + h * CH, CH)])
        plsc.subcore_barrier()

        def deg_body(b, _):
            pltpu.sync_copy(gfat, acc_s.at[col_v.at[b]], add=True)
            return 0
        lax.fori_loop(0, NFB, deg_body, 0)
        plsc.subcore_barrier()

        for h in range(RPT // CH):
            pltpu.sync_copy(acc_s.at[pl.ds(base + h * CH, CH)], buf)

            def deg_tab(r, _):
                d = jnp.maximum(buf[r, pl.ds(0, 16)], 1.0)
                dinvb[r + h * CH] = 0.5 / d
                return 0
            lax.fori_loop(0, CH, deg_tab, 0)

        def init_state(load_chunk, publish_s0):
            for h in range(RPT // CH):
                r0 = base + h * CH
                load_chunk(h, r0)
                pltpu.sync_copy(buf, xst_hbm.at[pl.ds(coff + r0, CH)])
                if publish_s0:
                    pltpu.sync_copy(buf, feat_hbm.at[0, c, pl.ds(r0, CH)])

                def iscale(r, _):
                    dd = 0.5 / dinvb[h * CH + r]
                    for g in range(NG):
                        buf[r, pl.ds(g * 16, 16)] = (
                            buf[r, pl.ds(g * 16, 16)] * dd)
                    return 0
                lax.fori_loop(0, CH, iscale, 0)
                pltpu.sync_copy(buf, acc_s.at[pl.ds(r0, CH)])
            plsc.subcore_barrier()

        def gather_scatter():
            def body(j, _):
                pltpu.async_copy(xst_hbm.at[row_v.at[j]], gfat,
                                 gsem.at[0]).wait()
                pltpu.async_copy(gfat, acc_s.at[col_v.at[j]],
                                 ssem.at[0], add=True).wait()
                return 0
            lax.fori_loop(0, NFB, body, 0)

        def update_io(t, stores, emits):
            for h in range(RPT // CH):
                r0 = base + h * CH
                pltpu.sync_copy(acc_s.at[pl.ds(r0, CH)], buf)

                def ub(r, _):
                    dv = dinvb[h * CH + r]
                    for g in range(NG):
                        av = buf[r, pl.ds(g * 16, 16)]
                        buf2[r, pl.ds(g * 16, 16)] = 0.5 * av
                        buf[r, pl.ds(g * 16, 16)] = dv * av
                    return 0
                lax.fori_loop(0, CH, ub, 0)
                pltpu.sync_copy(buf, xst_hbm.at[pl.ds(coff + r0, CH)])
                for tt, slot in stores.items():
                    @pl.when(t == tt)
                    def _():
                        pltpu.sync_copy(buf, feat_hbm.at[slot, c, pl.ds(r0, CH)])
                for tt, slot in emits.items():
                    @pl.when(t == tt)
                    def _():
                        pltpu.sync_copy(feat_hbm.at[slot, c, pl.ds(r0, CH)], ebuf)

                        def eb(i, _):
                            r = i // NG
                            g = i % NG
                            d = (buf[r, pl.ds(g * 16, 16)]
                                 - ebuf[r, pl.ds(g * 16, 16)])
                            ebuf[r, pl.ds(g * 16, 16)] = jnp.abs(d)
                            return 0
                        lax.fori_loop(0, CH * NG, eb, 0)
                        pltpu.sync_copy(ebuf, feat_hbm.at[slot, c, pl.ds(r0, CH)])
                pltpu.sync_copy(buf2, acc_s.at[pl.ds(r0, CH)])

        def run_stage(stores, emits):
            def step(t, _):
                gather_scatter()
                plsc.subcore_barrier()
                update_io(t, stores, emits)
                plsc.subcore_barrier()
                return 0
            lax.fori_loop(1, 17, step, 0)

        def load_x(h, r0):
            pltpu.sync_copy(x_hbm.at[c, pl.ds(r0, CH)], buf)
        init_state(load_x, True)
        run_stage(STAGE1_STORES, STAGE1_EMITS)

        for j1 in range(3):
            def load_s1(h, r0, j1=j1):
                pltpu.sync_copy(feat_hbm.at[1 + j1, c, pl.ds(r0, CH)], buf)
            init_state(load_s1, False)
            run_stage(STAGE2_STORES[j1], STAGE2_EMITS[j1])

    feat, _ = k(x_pad, epad)
    return feat


def _tc_moments(feat):
    def body(f_ref, o_ref):
        d = f_ref[0, 0]
        rows = lax.broadcasted_iota(jnp.int32, (NPAD, HC), 0)
        mask = rows < N
        dm = jnp.where(mask, d, 0.0)
        mean = jnp.sum(dm, axis=0) / N
        dev = jnp.where(mask, d - mean[None, :], 0.0)
        d2 = dev * dev
        var = jnp.sum(d2, axis=0) / N
        m3 = jnp.sum(d2 * dev, axis=0) / N
        m4 = jnp.sum(d2 * d2, axis=0) / N
        skew = m3 / var ** 1.5
        skew = jnp.where(skew > 1e15, 0.0, skew)
        skew = jnp.where(jnp.isnan(skew), 0.0, skew)
        kurt = m4 / (var * var) - 3.0
        kurt = jnp.where(kurt > 1e15, -3.0, kurt)
        kurt = jnp.where(jnp.isnan(kurt), -3.0, kurt)
        o_ref[0, 0] = jnp.stack([mean, var, skew, kurt])

    return pl.pallas_call(
        body,
        grid=(NSLOT * NCORE,),
        in_specs=[pl.BlockSpec((1, 1, NPAD, HC), lambda i: (i // 2, i % 2, 0, 0))],
        out_specs=pl.BlockSpec((1, 1, 4, HC), lambda i: (i // 2, i % 2, 0, 0)),
        out_shape=jax.ShapeDtypeStruct((NSLOT, NCORE, 4, HC), jnp.float32),
    )(feat)


def kernel(x, edge_index, wavelet_constructor):
    e = edge_index.reshape(2, NSUB, EPT)
    pad = jnp.concatenate([
        jnp.zeros((1, NSUB, EPAD - EPT), jnp.int32),
        jnp.full((1, NSUB, EPAD - EPT), PAD_NODE, jnp.int32),
    ], axis=0)
    epad = jnp.concatenate([e, pad], axis=2).reshape(2, NSUB, NFB, FB)
    x_pad = (jnp.zeros((NPAD, C), jnp.float32).at[:N].set(x)
             .reshape(NPAD, NCORE, HC).transpose(1, 0, 2))

    feat = _sc_scatter(x_pad, epad)
    stats = _tc_moments(feat)
    moments = jnp.transpose(stats, (2, 0, 1, 3)).reshape(1, 4 * NSLOT * C)
    return (moments, wavelet_constructor)


# speedup vs baseline: 1.2926x; 1.2926x over previous
import functools

import jax
import jax.numpy as jnp
from jax import lax
from jax.experimental import pallas as pl
from jax.experimental.pallas import tpu as pltpu
from jax.experimental.pallas import tpu_sc as plsc

N = 10000
E = 160000
C = 128

NSUB = 16
NCORE = 2
NPAD = 10240
RPT = NPAD // NSUB
HC = C // NCORE
EPT = E // NSUB
BATCH = 128
NB = (EPT + BATCH - 1) // BATCH
EPAD = 10240
NB = EPAD // BATCH
PAD_NODE = NPAD - 1
NSLOT = 11
CH = 128
NBUF = 4
FB = 2 * BATCH
NFB = NB // 2

STAGE1_STORES = {1: 1, 2: 2, 4: 3, 8: 4}
STAGE1_EMITS = {2: 1, 4: 2, 8: 3, 16: 4}
STAGE2_STORES = ({2: 5, 4: 6, 8: 8}, {4: 7, 8: 9}, {8: 10})
STAGE2_EMITS = ({4: 5, 8: 6, 16: 8}, {8: 7, 16: 9}, {16: 10})


def _sc_scatter(x_pad, epad):
    mesh = plsc.VectorSubcoreMesh(core_axis_name="c", subcore_axis_name="s")

    @functools.partial(
        pl.kernel,
        out_type=[
            jax.ShapeDtypeStruct((NSLOT, NCORE, NPAD, HC), jnp.float32),
            jax.ShapeDtypeStruct((NCORE * NPAD, HC), jnp.float32),
        ],
        mesh=mesh,
        compiler_params=pltpu.CompilerParams(use_tc_tiling_on_sc=False),
        scratch_types=[
            pltpu.VMEM_SHARED((NPAD, HC), jnp.float32),
            pltpu.VMEM((NFB, FB), jnp.int32),
            pltpu.VMEM((NFB, FB), jnp.int32),
            pltpu.VMEM((2, FB, HC), jnp.float32),
            pltpu.VMEM((CH, HC), jnp.float32),
            pltpu.VMEM((CH, HC), jnp.float32),
            pltpu.VMEM((CH, HC), jnp.float32),
            pltpu.VMEM((RPT, 16), jnp.float32),
            pltpu.SemaphoreType.DMA((NBUF,)),
            pltpu.SemaphoreType.DMA((NBUF,)),
        ],
    )
    def k(x_hbm, e_hbm, feat_hbm, xst_hbm, acc_s,
          row_v, col_v, gfat, buf, buf2, ebuf, dinvb, gsem, ssem):
        c = lax.axis_index("c")
        s = lax.axis_index("s")
        base = s * RPT
        NG = HC // 16
        zero16 = jnp.zeros((16,), jnp.float32)
        one16 = jnp.ones((16,), jnp.float32)

        pltpu.sync_copy(e_hbm.at[0, s], row_v)
        pltpu.sync_copy(e_hbm.at[1, s], col_v)

        coff = c * NPAD

        def off_body(i, _):
            r = i // (FB // 16)
            g = i % (FB // 16)
            row_v[r, pl.ds(g * 16, 16)] = row_v[r, pl.ds(g * 16, 16)] + coff
            return 0
        lax.fori_loop(0, NFB * (FB // 16), off_body, 0)

        gf0 = gfat.at[0]

        def fill_ones(i, _):
            gf0[i // NG, pl.ds((i % NG) * 16, 16)] = one16
            return 0
        lax.fori_loop(0, FB * NG, fill_ones, 0)

        def fill_zero(i, _):
            buf[i // NG, pl.ds((i % NG) * 16, 16)] = zero16
            return 0
        lax.fori_loop(0, CH * NG, fill_zero, 0)
        for h in range(RPT // CH):
            pltpu.sync_copy(buf, acc_s.at[pl.ds(base + h * CH, CH)])
        plsc.subcore_barrier()

        def deg_body(b, _):
            pltpu.sync_copy(gf0, acc_s.at[col_v.at[b]], add=True)
            return 0
        lax.fori_loop(0, NFB, deg_body, 0)
        plsc.subcore_barrier()

        for h in range(RPT // CH):
            pltpu.sync_copy(acc_s.at[pl.ds(base + h * CH, CH)], buf)

            def deg_tab(r, _):
                d = jnp.maximum(buf[r, pl.ds(0, 16)], 1.0)
                dinvb[r + h * CH] = 0.5 / d
                return 0
            lax.fori_loop(0, CH, deg_tab, 0)

        def init_state(load_chunk, publish_s0):
            for h in range(RPT // CH):
                r0 = base + h * CH
                load_chunk(h, r0)
                pltpu.sync_copy(buf, xst_hbm.at[pl.ds(coff + r0, CH)])
                if publish_s0:
                    pltpu.sync_copy(buf, feat_hbm.at[0, c, pl.ds(r0, CH)])

                def iscale(r, _):
                    dd = 0.5 / dinvb[h * CH + r]
                    for g in range(NG):
                        buf[r, pl.ds(g * 16, 16)] = (
                            buf[r, pl.ds(g * 16, 16)] * dd)
                    return 0
                lax.fori_loop(0, CH, iscale, 0)
                pltpu.sync_copy(buf, acc_s.at[pl.ds(r0, CH)])
            plsc.subcore_barrier()

        def start_g(b, kk):
            pltpu.async_copy(xst_hbm.at[row_v.at[b]], gfat.at[kk],
                             gsem.at[kk])

        def wait_g(b, kk):
            pltpu.make_async_copy(xst_hbm.at[row_v.at[b]], gfat.at[kk],
                                  gsem.at[kk]).wait()

        def gather_scatter():
            start_g(0, 0)
            start_g(1, 1)

            def body(j, _):
                for kk in range(2):
                    b = j * 2 + kk
                    wait_g(b, kk)
                    pltpu.async_copy(gfat.at[kk], acc_s.at[col_v.at[b]],
                                     ssem.at[kk], add=True).wait()

                    @pl.when(b + 2 < NFB)
                    def _():
                        start_g(b + 2, kk)
                return 0
            lax.fori_loop(0, NFB // 2, body, 0)

        def update_io(t, stores, emits):
            for h in range(RPT // CH):
                r0 = base + h * CH
                pltpu.sync_copy(acc_s.at[pl.ds(r0, CH)], buf)

                def ub(r, _):
                    dv = dinvb[h * CH + r]
                    for g in range(NG):
                        av = buf[r, pl.ds(g * 16, 16)]
                        buf2[r, pl.ds(g * 16, 16)] = 0.5 * av
                        buf[r, pl.ds(g * 16, 16)] = dv * av
                    return 0
                lax.fori_loop(0, CH, ub, 0)
                pltpu.sync_copy(buf, xst_hbm.at[pl.ds(coff + r0, CH)])
                for tt, slot in stores.items():
                    @pl.when(t == tt)
                    def _():
                        pltpu.sync_copy(buf, feat_hbm.at[slot, c, pl.ds(r0, CH)])
                for tt, slot in emits.items():
                    @pl.when(t == tt)
                    def _():
                        pltpu.sync_copy(feat_hbm.at[slot, c, pl.ds(r0, CH)], ebuf)

                        def eb(i, _):
                            r = i // NG
                            g = i % NG
                            d = (buf[r, pl.ds(g * 16, 16)]
                                 - ebuf[r, pl.ds(g * 16, 16)])
                            ebuf[r, pl.ds(g * 16, 16)] = jnp.abs(d)
                            return 0
                        lax.fori_loop(0, CH * NG, eb, 0)
                        pltpu.sync_copy(ebuf, feat_hbm.at[slot, c, pl.ds(r0, CH)])
                pltpu.sync_copy(buf2, acc_s.at[pl.ds(r0, CH)])

        def run_stage(stores, emits):
            def step(t, _):
                gather_scatter()
                plsc.subcore_barrier()
                update_io(t, stores, emits)
                plsc.subcore_barrier()
                return 0
            lax.fori_loop(1, 17, step, 0)

        def load_x(h, r0):
            pltpu.sync_copy(x_hbm.at[c, pl.ds(r0, CH)], buf)
        init_state(load_x, True)
        run_stage(STAGE1_STORES, STAGE1_EMITS)

        for j1 in range(3):
            def load_s1(h, r0, j1=j1):
                pltpu.sync_copy(feat_hbm.at[1 + j1, c, pl.ds(r0, CH)], buf)
            init_state(load_s1, False)
            run_stage(STAGE2_STORES[j1], STAGE2_EMITS[j1])

    feat, _ = k(x_pad, epad)
    return feat


def _tc_moments(feat):
    def body(f_ref, o_ref):
        d = f_ref[0, 0]
        rows = lax.broadcasted_iota(jnp.int32, (NPAD, HC), 0)
        mask = rows < N
        dm = jnp.where(mask, d, 0.0)
        mean = jnp.sum(dm, axis=0) / N
        dev = jnp.where(mask, d - mean[None, :], 0.0)
        d2 = dev * dev
        var = jnp.sum(d2, axis=0) / N
        m3 = jnp.sum(d2 * dev, axis=0) / N
        m4 = jnp.sum(d2 * d2, axis=0) / N
        skew = m3 / var ** 1.5
        skew = jnp.where(skew > 1e15, 0.0, skew)
        skew = jnp.where(jnp.isnan(skew), 0.0, skew)
        kurt = m4 / (var * var) - 3.0
        kurt = jnp.where(kurt > 1e15, -3.0, kurt)
        kurt = jnp.where(jnp.isnan(kurt), -3.0, kurt)
        o_ref[0, 0] = jnp.stack([mean, var, skew, kurt])

    return pl.pallas_call(
        body,
        grid=(NSLOT * NCORE,),
        in_specs=[pl.BlockSpec((1, 1, NPAD, HC), lambda i: (i // 2, i % 2, 0, 0))],
        out_specs=pl.BlockSpec((1, 1, 4, HC), lambda i: (i // 2, i % 2, 0, 0)),
        out_shape=jax.ShapeDtypeStruct((NSLOT, NCORE, 4, HC), jnp.float32),
    )(feat)


def kernel(x, edge_index, wavelet_constructor):
    e = edge_index.reshape(2, NSUB, EPT)
    pad = jnp.concatenate([
        jnp.zeros((1, NSUB, EPAD - EPT), jnp.int32),
        jnp.full((1, NSUB, EPAD - EPT), PAD_NODE, jnp.int32),
    ], axis=0)
    epad = jnp.concatenate([e, pad], axis=2).reshape(2, NSUB, NFB, FB)
    x_pad = (jnp.zeros((NPAD, C), jnp.float32).at[:N].set(x)
             .reshape(NPAD, NCORE, HC).transpose(1, 0, 2))

    feat = _sc_scatter(x_pad, epad)
    stats = _tc_moments(feat)
    moments = jnp.transpose(stats, (2, 0, 1, 3)).reshape(1, 4 * NSLOT * C)
    return (moments, wavelet_constructor)


# speedup vs baseline: 1.4347x; 1.1099x over previous
import functools

import jax
import jax.numpy as jnp
from jax import lax
from jax.experimental import pallas as pl
from jax.experimental.pallas import tpu as pltpu
from jax.experimental.pallas import tpu_sc as plsc

N = 10000
E = 160000
C = 128

NSUB = 16
NCORE = 2
NPAD = 10240
RPT = NPAD // NSUB
HC = C // NCORE
EPT = E // NSUB
BATCH = 128
NB = (EPT + BATCH - 1) // BATCH
EPAD = 10240
NB = EPAD // BATCH
PAD_NODE = NPAD - 1
NSLOT = 11
CH = 128
NBUF = 4
FB = BATCH
NFB = NB

STAGE1_STORES = {1: 1, 2: 2, 4: 3, 8: 4}
STAGE1_EMITS = {2: 1, 4: 2, 8: 3, 16: 4}
STAGE2_STORES = ({2: 5, 4: 6, 8: 8}, {4: 7, 8: 9}, {8: 10})
STAGE2_EMITS = ({4: 5, 8: 6, 16: 8}, {8: 7, 16: 9}, {16: 10})


def _sc_scatter(x_pad, epad):
    mesh = plsc.VectorSubcoreMesh(core_axis_name="c", subcore_axis_name="s")

    @functools.partial(
        pl.kernel,
        out_type=[
            jax.ShapeDtypeStruct((NSLOT, NCORE, NPAD, HC), jnp.float32),
            jax.ShapeDtypeStruct((NCORE * NPAD, HC), jnp.float32),
        ],
        mesh=mesh,
        compiler_params=pltpu.CompilerParams(use_tc_tiling_on_sc=False),
        scratch_types=[
            pltpu.VMEM_SHARED((NPAD, HC), jnp.float32),
            pltpu.VMEM((NFB, FB), jnp.int32),
            pltpu.VMEM((NFB, FB), jnp.int32),
            pltpu.VMEM((NBUF, FB, HC), jnp.float32),
            pltpu.VMEM((CH, HC), jnp.float32),
            pltpu.VMEM((CH, HC), jnp.float32),
            pltpu.VMEM((CH, HC), jnp.float32),
            pltpu.VMEM((RPT, 16), jnp.float32),
            pltpu.SemaphoreType.DMA((NBUF,)),
            pltpu.SemaphoreType.DMA((NBUF,)),
        ],
    )
    def k(x_hbm, e_hbm, feat_hbm, xst_hbm, acc_s,
          row_v, col_v, gfat, buf, buf2, ebuf, dinvb, gsem, ssem):
        c = lax.axis_index("c")
        s = lax.axis_index("s")
        base = s * RPT
        NG = HC // 16
        zero16 = jnp.zeros((16,), jnp.float32)
        one16 = jnp.ones((16,), jnp.float32)

        pltpu.sync_copy(e_hbm.at[0, s], row_v)
        pltpu.sync_copy(e_hbm.at[1, s], col_v)

        coff = c * NPAD

        def off_body(i, _):
            r = i // (FB // 16)
            g = i % (FB // 16)
            row_v[r, pl.ds(g * 16, 16)] = row_v[r, pl.ds(g * 16, 16)] + coff
            return 0
        lax.fori_loop(0, NFB * (FB // 16), off_body, 0)

        gf0 = gfat.at[0]

        def fill_ones(i, _):
            gf0[i // NG, pl.ds((i % NG) * 16, 16)] = one16
            return 0
        lax.fori_loop(0, FB * NG, fill_ones, 0)

        def fill_zero(i, _):
            buf[i // NG, pl.ds((i % NG) * 16, 16)] = zero16
            return 0
        lax.fori_loop(0, CH * NG, fill_zero, 0)
        for h in range(RPT // CH):
            pltpu.sync_copy(buf, acc_s.at[pl.ds(base + h * CH, CH)])
        plsc.subcore_barrier()

        def deg_body(b, _):
            pltpu.sync_copy(gf0, acc_s.at[col_v.at[b]], add=True)
            return 0
        lax.fori_loop(0, NFB, deg_body, 0)
        plsc.subcore_barrier()

        for h in range(RPT // CH):
            pltpu.sync_copy(acc_s.at[pl.ds(base + h * CH, CH)], buf)

            def deg_tab(r, _):
                d = jnp.maximum(buf[r, pl.ds(0, 16)], 1.0)
                dinvb[r + h * CH] = 0.5 / d
                return 0
            lax.fori_loop(0, CH, deg_tab, 0)

        def init_state(load_chunk, publish_s0):
            for h in range(RPT // CH):
                r0 = base + h * CH
                load_chunk(h, r0)
                pltpu.sync_copy(buf, xst_hbm.at[pl.ds(coff + r0, CH)])
                if publish_s0:
                    pltpu.sync_copy(buf, feat_hbm.at[0, c, pl.ds(r0, CH)])

                def iscale(r, _):
                    dd = 0.5 / dinvb[h * CH + r]
                    for g in range(NG):
                        buf[r, pl.ds(g * 16, 16)] = (
                            buf[r, pl.ds(g * 16, 16)] * dd)
                    return 0
                lax.fori_loop(0, CH, iscale, 0)
                pltpu.sync_copy(buf, acc_s.at[pl.ds(r0, CH)])
            plsc.subcore_barrier()

        def start_g(b, kk):
            pltpu.async_copy(xst_hbm.at[row_v.at[b]], gfat.at[kk],
                             gsem.at[kk])

        def wait_g(b, kk):
            pltpu.make_async_copy(xst_hbm.at[row_v.at[b]], gfat.at[kk],
                                  gsem.at[kk]).wait()

        def gather_scatter():
            for kk in range(NBUF):
                start_g(kk, kk)

            def body(j, _):
                for kk in range(NBUF):
                    b = j * NBUF + kk
                    wait_g(b, kk)
                    pltpu.async_copy(gfat.at[kk], acc_s.at[col_v.at[b]],
                                     ssem.at[kk], add=True).wait()

                    @pl.when(b + NBUF < NFB)
                    def _():
                        start_g(b + NBUF, kk)
                return 0
            lax.fori_loop(0, NFB // NBUF, body, 0)

        def update_io(t, stores, emits):
            for h in range(RPT // CH):
                r0 = base + h * CH
                pltpu.sync_copy(acc_s.at[pl.ds(r0, CH)], buf)

                def ub(r, _):
                    dv = dinvb[h * CH + r]
                    for g in range(NG):
                        av = buf[r, pl.ds(g * 16, 16)]
                        buf2[r, pl.ds(g * 16, 16)] = 0.5 * av
                        buf[r, pl.ds(g * 16, 16)] = dv * av
                    return 0
                lax.fori_loop(0, CH, ub, 0)
                pltpu.sync_copy(buf, xst_hbm.at[pl.ds(coff + r0, CH)])
                for tt, slot in stores.items():
                    @pl.when(t == tt)
                    def _():
                        pltpu.sync_copy(buf, feat_hbm.at[slot, c, pl.ds(r0, CH)])
                for tt, slot in emits.items():
                    @pl.when(t == tt)
                    def _():
                        pltpu.sync_copy(feat_hbm.at[slot, c, pl.ds(r0, CH)], ebuf)

                        def eb(i, _):
                            r = i // NG
                            g = i % NG
                            d = (buf[r, pl.ds(g * 16, 16)]
                                 - ebuf[r, pl.ds(g * 16, 16)])
                            ebuf[r, pl.ds(g * 16, 16)] = jnp.abs(d)
                            return 0
                        lax.fori_loop(0, CH * NG, eb, 0)
                        pltpu.sync_copy(ebuf, feat_hbm.at[slot, c, pl.ds(r0, CH)])
                pltpu.sync_copy(buf2, acc_s.at[pl.ds(r0, CH)])

        def run_stage(stores, emits):
            def step(t, _):
                gather_scatter()
                plsc.subcore_barrier()
                update_io(t, stores, emits)
                plsc.subcore_barrier()
                return 0
            lax.fori_loop(1, 17, step, 0)

        def load_x(h, r0):
            pltpu.sync_copy(x_hbm.at[c, pl.ds(r0, CH)], buf)
        init_state(load_x, True)
        run_stage(STAGE1_STORES, STAGE1_EMITS)

        for j1 in range(3):
            def load_s1(h, r0, j1=j1):
                pltpu.sync_copy(feat_hbm.at[1 + j1, c, pl.ds(r0, CH)], buf)
            init_state(load_s1, False)
            run_stage(STAGE2_STORES[j1], STAGE2_EMITS[j1])

    feat, _ = k(x_pad, epad)
    return feat


def _tc_moments(feat):
    def body(f_ref, o_ref):
        d = f_ref[0, 0]
        rows = lax.broadcasted_iota(jnp.int32, (NPAD, HC), 0)
        mask = rows < N
        dm = jnp.where(mask, d, 0.0)
        mean = jnp.sum(dm, axis=0) / N
        dev = jnp.where(mask, d - mean[None, :], 0.0)
        d2 = dev * dev
        var = jnp.sum(d2, axis=0) / N
        m3 = jnp.sum(d2 * dev, axis=0) / N
        m4 = jnp.sum(d2 * d2, axis=0) / N
        skew = m3 / var ** 1.5
        skew = jnp.where(skew > 1e15, 0.0, skew)
        skew = jnp.where(jnp.isnan(skew), 0.0, skew)
        kurt = m4 / (var * var) - 3.0
        kurt = jnp.where(kurt > 1e15, -3.0, kurt)
        kurt = jnp.where(jnp.isnan(kurt), -3.0, kurt)
        o_ref[0, 0] = jnp.stack([mean, var, skew, kurt])

    return pl.pallas_call(
        body,
        grid=(NSLOT * NCORE,),
        in_specs=[pl.BlockSpec((1, 1, NPAD, HC), lambda i: (i // 2, i % 2, 0, 0))],
        out_specs=pl.BlockSpec((1, 1, 4, HC), lambda i: (i // 2, i % 2, 0, 0)),
        out_shape=jax.ShapeDtypeStruct((NSLOT, NCORE, 4, HC), jnp.float32),
    )(feat)


def kernel(x, edge_index, wavelet_constructor):
    e = edge_index.reshape(2, NSUB, EPT)
    pad = jnp.concatenate([
        jnp.zeros((1, NSUB, EPAD - EPT), jnp.int32),
        jnp.full((1, NSUB, EPAD - EPT), PAD_NODE, jnp.int32),
    ], axis=0)
    epad = jnp.concatenate([e, pad], axis=2).reshape(2, NSUB, NFB, FB)
    x_pad = (jnp.zeros((NPAD, C), jnp.float32).at[:N].set(x)
             .reshape(NPAD, NCORE, HC).transpose(1, 0, 2))

    feat = _sc_scatter(x_pad, epad)
    stats = _tc_moments(feat)
    moments = jnp.transpose(stats, (2, 0, 1, 3)).reshape(1, 4 * NSLOT * C)
    return (moments, wavelet_constructor)


# speedup vs baseline: 1.4663x; 1.0220x over previous
import functools

import jax
import jax.numpy as jnp
from jax import lax
from jax.experimental import pallas as pl
from jax.experimental.pallas import tpu as pltpu
from jax.experimental.pallas import tpu_sc as plsc

N = 10000
E = 160000
C = 128

NSUB = 16
NCORE = 2
NPAD = 10240
RPT = NPAD // NSUB
HC = C // NCORE
EPT = E // NSUB
BATCH = 128
NB = (EPT + BATCH - 1) // BATCH
EPAD = 10240
NB = EPAD // BATCH
PAD_NODE = NPAD - 1
NSLOT = 11
CH = 128
NBUF = 4
FB = BATCH
NFB = NB

STAGE1_STORES = {1: 1, 2: 2, 4: 3, 8: 4}
STAGE1_EMITS = {2: 1, 4: 2, 8: 3, 16: 4}
STAGE2_STORES = ({2: 5, 4: 6, 8: 8}, {4: 7, 8: 9}, {8: 10})
STAGE2_EMITS = ({4: 5, 8: 6, 16: 8}, {8: 7, 16: 9}, {16: 10})


def _sc_scatter(x_pad, epad):
    mesh = plsc.VectorSubcoreMesh(core_axis_name="c", subcore_axis_name="s")

    @functools.partial(
        pl.kernel,
        out_type=[
            jax.ShapeDtypeStruct((NSLOT, NCORE, NPAD, HC), jnp.float32),
            jax.ShapeDtypeStruct((NCORE * NPAD, HC), jnp.float32),
        ],
        mesh=mesh,
        compiler_params=pltpu.CompilerParams(use_tc_tiling_on_sc=False),
        scratch_types=[
            pltpu.VMEM_SHARED((NPAD, HC), jnp.float32),
            pltpu.VMEM((NFB, FB), jnp.int32),
            pltpu.VMEM((NFB, FB), jnp.int32),
            pltpu.VMEM((NBUF, FB, HC), jnp.float32),
            pltpu.VMEM((CH, HC), jnp.float32),
            pltpu.VMEM((CH, HC), jnp.float32),
            pltpu.VMEM((CH, HC), jnp.float32),
            pltpu.VMEM((RPT, 16), jnp.float32),
            pltpu.SemaphoreType.DMA((NBUF,)),
            pltpu.SemaphoreType.DMA((NBUF,)),
        ],
    )
    def k(x_hbm, e_hbm, feat_hbm, xst_hbm, acc_s,
          row_v, col_v, gfat, buf, buf2, ebuf, dinvb, gsem, ssem):
        c = lax.axis_index("c")
        s = lax.axis_index("s")
        base = s * RPT
        NG = HC // 16
        zero16 = jnp.zeros((16,), jnp.float32)
        one16 = jnp.ones((16,), jnp.float32)

        pltpu.sync_copy(e_hbm.at[0, s], row_v)
        pltpu.sync_copy(e_hbm.at[1, s], col_v)

        coff = c * NPAD

        def off_body(i, _):
            r = i // (FB // 16)
            g = i % (FB // 16)
            row_v[r, pl.ds(g * 16, 16)] = row_v[r, pl.ds(g * 16, 16)] + coff
            return 0
        lax.fori_loop(0, NFB * (FB // 16), off_body, 0)

        gf0 = gfat.at[0]

        def fill_ones(i, _):
            gf0[i // NG, pl.ds((i % NG) * 16, 16)] = one16
            return 0
        lax.fori_loop(0, FB * NG, fill_ones, 0)

        def fill_zero(i, _):
            buf[i // NG, pl.ds((i % NG) * 16, 16)] = zero16
            return 0
        lax.fori_loop(0, CH * NG, fill_zero, 0)
        for h in range(RPT // CH):
            pltpu.sync_copy(buf, acc_s.at[pl.ds(base + h * CH, CH)])
        plsc.subcore_barrier()

        def deg_body(b, _):
            pltpu.sync_copy(gf0, acc_s.at[col_v.at[b]], add=True)
            return 0
        lax.fori_loop(0, NFB, deg_body, 0)
        plsc.subcore_barrier()

        for h in range(RPT // CH):
            pltpu.sync_copy(acc_s.at[pl.ds(base + h * CH, CH)], buf)

            def deg_tab(r, _):
                d = jnp.maximum(buf[r, pl.ds(0, 16)], 1.0)
                dinvb[r + h * CH] = 0.5 / d
                return 0
            lax.fori_loop(0, CH, deg_tab, 0)

        def init_state(load_chunk, publish_s0):
            for h in range(RPT // CH):
                r0 = base + h * CH
                load_chunk(h, r0)
                pltpu.sync_copy(buf, xst_hbm.at[pl.ds(coff + r0, CH)])
                if publish_s0:
                    pltpu.sync_copy(buf, feat_hbm.at[0, c, pl.ds(r0, CH)])

                def iscale(r, _):
                    dd = 0.5 / dinvb[h * CH + r]
                    for g in range(NG):
                        buf[r, pl.ds(g * 16, 16)] = (
                            buf[r, pl.ds(g * 16, 16)] * dd)
                    return 0
                lax.fori_loop(0, CH, iscale, 0)
                pltpu.sync_copy(buf, acc_s.at[pl.ds(r0, CH)])
            plsc.subcore_barrier()

        def start_g(b, kk):
            pltpu.async_copy(xst_hbm.at[row_v.at[b]], gfat.at[kk],
                             gsem.at[kk])

        def wait_g(b, kk):
            pltpu.make_async_copy(xst_hbm.at[row_v.at[b]], gfat.at[kk],
                                  gsem.at[kk]).wait()

        def gather_scatter():
            for kk in range(NBUF):
                start_g(kk, kk)

            def body(j, _):
                for kk in range(NBUF):
                    b = j * NBUF + kk
                    wait_g(b, kk)
                    pltpu.async_copy(gfat.at[kk], acc_s.at[col_v.at[b]],
                                     ssem.at[kk], add=True).wait()

                    @pl.when(b + NBUF < NFB)
                    def _():
                        start_g(b + NBUF, kk)
                return 0
            lax.fori_loop(0, NFB // NBUF, body, 0)

        def update_io(t, stores, emits):
            mains = [buf, gfat.at[0]]
            seeds = [buf2, gfat.at[1]]

            def accin(h, start):
                m = mains[h % 2]
                cp = (pltpu.async_copy if start else pltpu.make_async_copy)(
                    acc_s.at[pl.ds(base + h * CH, CH)], m, gsem.at[h % 2])
                if not start:
                    cp.wait()

            def xout(h, start):
                m = mains[h % 2]
                cp = (pltpu.async_copy if start else pltpu.make_async_copy)(
                    m, xst_hbm.at[pl.ds(coff + base + h * CH, CH)],
                    ssem.at[h % 2])
                if not start:
                    cp.wait()

            def accout(h, start):
                sd = seeds[h % 2]
                cp = (pltpu.async_copy if start else pltpu.make_async_copy)(
                    sd, acc_s.at[pl.ds(base + h * CH, CH)],
                    ssem.at[2 + h % 2])
                if not start:
                    cp.wait()

            NCHUNK = RPT // CH
            accin(0, True)
            for h in range(NCHUNK):
                m = mains[h % 2]
                sd = seeds[h % 2]
                r0 = base + h * CH
                if h >= 1:
                    xout(h - 1, False)
                    accout(h - 1, False)
                if h + 1 < NCHUNK:
                    accin(h + 1, True)
                accin(h, False)

                def ub(r, _):
                    dv = dinvb[h * CH + r]
                    for g in range(NG):
                        av = m[r, pl.ds(g * 16, 16)]
                        sd[r, pl.ds(g * 16, 16)] = 0.5 * av
                        m[r, pl.ds(g * 16, 16)] = dv * av
                    return 0
                lax.fori_loop(0, CH, ub, 0)
                xout(h, True)
                accout(h, True)
                for tt, slot in stores.items():
                    @pl.when(t == tt)
                    def _():
                        pltpu.sync_copy(m, feat_hbm.at[slot, c, pl.ds(r0, CH)])
                for tt, slot in emits.items():
                    @pl.when(t == tt)
                    def _():
                        pltpu.sync_copy(feat_hbm.at[slot, c, pl.ds(r0, CH)], ebuf)

                        def eb(i, _):
                            r = i // NG
                            g = i % NG
                            d = (m[r, pl.ds(g * 16, 16)]
                                 - ebuf[r, pl.ds(g * 16, 16)])
                            ebuf[r, pl.ds(g * 16, 16)] = jnp.abs(d)
                            return 0
                        lax.fori_loop(0, CH * NG, eb, 0)
                        pltpu.sync_copy(ebuf, feat_hbm.at[slot, c, pl.ds(r0, CH)])
            xout(NCHUNK - 1, False)
            accout(NCHUNK - 1, False)

        def run_stage(stores, emits):
            def step(t, _):
                gather_scatter()
                plsc.subcore_barrier()
                update_io(t, stores, emits)
                plsc.subcore_barrier()
                return 0
            lax.fori_loop(1, 17, step, 0)

        def load_x(h, r0):
            pltpu.sync_copy(x_hbm.at[c, pl.ds(r0, CH)], buf)
        init_state(load_x, True)
        run_stage(STAGE1_STORES, STAGE1_EMITS)

        for j1 in range(3):
            def load_s1(h, r0, j1=j1):
                pltpu.sync_copy(feat_hbm.at[1 + j1, c, pl.ds(r0, CH)], buf)
            init_state(load_s1, False)
            run_stage(STAGE2_STORES[j1], STAGE2_EMITS[j1])

    feat, _ = k(x_pad, epad)
    return feat


def _tc_moments(feat):
    def body(f_ref, o_ref):
        d = f_ref[0, 0]
        rows = lax.broadcasted_iota(jnp.int32, (NPAD, HC), 0)
        mask = rows < N
        dm = jnp.where(mask, d, 0.0)
        mean = jnp.sum(dm, axis=0) / N
        dev = jnp.where(mask, d - mean[None, :], 0.0)
        d2 = dev * dev
        var = jnp.sum(d2, axis=0) / N
        m3 = jnp.sum(d2 * dev, axis=0) / N
        m4 = jnp.sum(d2 * d2, axis=0) / N
        skew = m3 / var ** 1.5
        skew = jnp.where(skew > 1e15, 0.0, skew)
        skew = jnp.where(jnp.isnan(skew), 0.0, skew)
        kurt = m4 / (var * var) - 3.0
        kurt = jnp.where(kurt > 1e15, -3.0, kurt)
        kurt = jnp.where(jnp.isnan(kurt), -3.0, kurt)
        o_ref[0, 0] = jnp.stack([mean, var, skew, kurt])

    return pl.pallas_call(
        body,
        grid=(NSLOT * NCORE,),
        in_specs=[pl.BlockSpec((1, 1, NPAD, HC), lambda i: (i // 2, i % 2, 0, 0))],
        out_specs=pl.BlockSpec((1, 1, 4, HC), lambda i: (i // 2, i % 2, 0, 0)),
        out_shape=jax.ShapeDtypeStruct((NSLOT, NCORE, 4, HC), jnp.float32),
    )(feat)


def kernel(x, edge_index, wavelet_constructor):
    e = edge_index.reshape(2, NSUB, EPT)
    pad = jnp.concatenate([
        jnp.zeros((1, NSUB, EPAD - EPT), jnp.int32),
        jnp.full((1, NSUB, EPAD - EPT), PAD_NODE, jnp.int32),
    ], axis=0)
    epad = jnp.concatenate([e, pad], axis=2).reshape(2, NSUB, NFB, FB)
    x_pad = (jnp.zeros((NPAD, C), jnp.float32).at[:N].set(x)
             .reshape(NPAD, NCORE, HC).transpose(1, 0, 2))

    feat = _sc_scatter(x_pad, epad)
    stats = _tc_moments(feat)
    moments = jnp.transpose(stats, (2, 0, 1, 3)).reshape(1, 4 * NSLOT * C)
    return (moments, wavelet_constructor)
